# baseline (speedup 1.0000x reference)
import functools
import jax
import jax.numpy as jnp
from jax import lax
from jax.experimental import pallas as pl
from jax.experimental.pallas import tpu as pltpu
from jax.experimental.pallas import tpu_sc as plsc

_N = 100000
_NC, _NS, _NW = 2, 16, 32
_CF = 625
_FCH_PER_W = 5
_CI = 400
_ICH = 250
_ICH_PER_W = 8

_mesh = plsc.VectorSubcoreMesh(core_axis_name="c", subcore_axis_name="s")


@functools.partial(
    pl.kernel,
    out_type=[
        jax.ShapeDtypeStruct((_N * 24,), jnp.int32),
        jax.ShapeDtypeStruct((_N, 8, 64), jnp.float32),
    ],
    mesh=_mesh,
    scratch_types=[
        pltpu.VMEM((_CF, 64), jnp.float32),
        pltpu.VMEM((_CI * 3,), jnp.int32),
        pltpu.VMEM((_CI * 24,), jnp.int32),
    ],
    compiler_params=pltpu.CompilerParams(
        use_tc_tiling_on_sc=False, needs_layout_passes=False),
)
def _sc_upsample(inds_hbm, feats_hbm, oinds_hbm, ofeats_hbm,
                 fbuf, ibuf, obuf):
    wid = lax.axis_index("s") * _NC + lax.axis_index("c")

    for t in range(_FCH_PER_W):
        base = (wid * _FCH_PER_W + t) * _CF
        pltpu.sync_copy(feats_hbm.at[pl.ds(base, _CF)], fbuf)
        for j in range(8):
            pltpu.sync_copy(fbuf, ofeats_hbm.at[pl.ds(base, _CF), j])

    lane = lax.iota(jnp.int32, 16)
    gidx, goff = [], []
    for s in range(3):
        m = lane + 16 * s
        i2 = (m * 2731) >> 16
        r24 = m - 24 * i2
        j = (r24 * 11) >> 5
        r3 = r24 - 3 * j
        gidx.append(3 * i2 + r3)
        msk = jnp.where(r3 == 0, 210, jnp.where(r3 == 1, 180, 232))
        goff.append((msk >> j) & 1)

    def do_chunk(k):
        pltpu.sync_copy(inds_hbm.at[pl.ds(k * _CI * 3, _CI * 3)], ibuf)

        def pair(p, carry):
            for s in range(3):
                v = plsc.load_gather(ibuf, [6 * p + gidx[s]])
                obuf[pl.ds(48 * p + 16 * s, 16)] = 2 * v + goff[s]
            return carry

        lax.fori_loop(0, _CI // 2, pair, 0)
        pltpu.sync_copy(obuf, oinds_hbm.at[pl.ds(k * _CI * 24, _CI * 24)])

    for t in range(_ICH_PER_W):
        k = wid + _NW * t

        @pl.when(k < _ICH)
        def _():
            do_chunk(k)


def kernel(voxel_inds, feats):
    oinds_flat, ofeats3 = _sc_upsample(voxel_inds.reshape(-1), feats)
    return oinds_flat.reshape(-1, 3), ofeats3.reshape(-1, 64)


# speedup vs baseline: 4.6043x; 4.6043x over previous
import math

import jax
import jax.numpy as jnp
import numpy as np
from jax.experimental import pallas as pl

_OFFS_NP = np.array(
    [[0, 0, 0], [1, 0, 0], [0, 1, 0], [0, 0, 1],
     [1, 1, 0], [0, 1, 1], [1, 0, 1], [1, 1, 1]], dtype=np.int32)

_N = 100000
_K = 16
_LI = 128 * _K
_LO = 1024 * _K
_GRID = math.ceil(_N / _LI)


def _body(offs_ref, indsT_ref, featsT_ref, oindsT_ref, ofeatsT_ref):
    x = featsT_ref[...]
    a = x.T
    b = jnp.broadcast_to(a[:, None, :], (_LI, 8, 64)).reshape(8 * _LI, 64)
    ofeatsT_ref[...] = b.T

    it = indsT_ref[...]
    ia = it.T
    ib = jnp.broadcast_to(ia[:, None, :] * 2, (_LI, 8, 3)) + offs_ref[...][None]
    oindsT_ref[...] = ib.reshape(8 * _LI, 3).T


def kernel(voxel_inds, feats):
    indsT = voxel_inds.T
    featsT = feats.T
    oindsT, ofeatsT = pl.pallas_call(
        _body,
        grid=(_GRID,),
        in_specs=[
            pl.BlockSpec((8, 3), lambda i: (0, 0)),
            pl.BlockSpec((3, _LI), lambda i: (0, i)),
            pl.BlockSpec((64, _LI), lambda i: (0, i)),
        ],
        out_specs=[
            pl.BlockSpec((3, _LO), lambda i: (0, i)),
            pl.BlockSpec((64, _LO), lambda i: (0, i)),
        ],
        out_shape=[
            jax.ShapeDtypeStruct((3, 8 * _N), jnp.int32),
            jax.ShapeDtypeStruct((64, 8 * _N), jnp.float32),
        ],
    )(jnp.asarray(_OFFS_NP), indsT, featsT)
    return oindsT.T, ofeatsT.T


# speedup vs baseline: 7.1637x; 1.5559x over previous
import functools
import math

import jax
import jax.numpy as jnp
import numpy as np
from jax import lax
from jax.experimental import pallas as pl
from jax.experimental.pallas import tpu as pltpu
from jax.experimental.pallas import tpu_sc as plsc

_N = 100000
_K = 16
_LI = 128 * _K
_LO = 1024 * _K
_GRID = math.ceil(_N / _LI)

_NC = 2
_WIN = 1000
_CHT = 100
_MASKS = (210, 180, 232)

_sc_mesh = plsc.VectorSubcoreMesh(core_axis_name="c", subcore_axis_name="s")


@functools.partial(
    pl.kernel,
    out_type=jax.ShapeDtypeStruct((3 * 8 * _N,), jnp.int32),
    mesh=_sc_mesh,
    scratch_types=[
        pltpu.VMEM((_WIN,), jnp.int32),
        pltpu.VMEM((8 * _WIN,), jnp.int32),
    ],
    compiler_params=pltpu.CompilerParams(
        use_tc_tiling_on_sc=False, needs_layout_passes=False),
)
def _sc_inds(indsT_hbm, oindsT_hbm, ibuf, obuf):
    wid = lax.axis_index("s") * _NC + lax.axis_index("c")
    lane = lax.iota(jnp.int32, 16)
    sel = lane >> 3
    j = lane & 7

    for c in range(3):
        off = (_MASKS[c] >> j) & 1
        for t in range(4):
            g = wid + 32 * t

            @pl.when(g < _CHT)
            def _():
                pltpu.sync_copy(
                    indsT_hbm.at[pl.ds(c * _N + g * _WIN, _WIN)], ibuf)

                def step(p, carry):
                    v = plsc.load_gather(ibuf, [2 * p + sel])
                    obuf[pl.ds(16 * p, 16)] = 2 * v + off
                    return carry

                lax.fori_loop(0, _WIN // 2, step, 0)
                pltpu.sync_copy(
                    obuf,
                    oindsT_hbm.at[pl.ds(c * 8 * _N + g * 8 * _WIN, 8 * _WIN)])


def _tc_body(featsT_ref, ofeatsT_ref):
    x = featsT_ref[...]
    a = x.T
    b = jnp.broadcast_to(a[:, None, :], (_LI, 8, 64)).reshape(8 * _LI, 64)
    ofeatsT_ref[...] = b.T


def kernel(voxel_inds, feats):
    indsT_flat = voxel_inds.T.reshape(-1)
    featsT = feats.T

    oindsT_flat = _sc_inds(indsT_flat)
    ofeatsT = pl.pallas_call(
        _tc_body,
        grid=(_GRID,),
        in_specs=[pl.BlockSpec((64, _LI), lambda i: (0, i))],
        out_specs=pl.BlockSpec((64, _LO), lambda i: (0, i)),
        out_shape=jax.ShapeDtypeStruct((64, 8 * _N), jnp.float32),
    )(featsT)

    return oindsT_flat.reshape(3, 8 * _N).T, ofeatsT.T


# speedup vs baseline: 9.5407x; 1.3318x over previous
import functools
import math

import jax
import jax.numpy as jnp
import numpy as np
from jax import lax
from jax.experimental import pallas as pl
from jax.experimental.pallas import tpu as pltpu
from jax.experimental.pallas import tpu_sc as plsc

_N = 100000
_K = 16
_LI = 128 * _K
_LO = 1024 * _K
_GRID = math.ceil(_N / _LI)

_NC = 2
_WIN = 1000
_CHT = 100
_MASKS = (210, 180, 232)

_sc_mesh = plsc.VectorSubcoreMesh(core_axis_name="c", subcore_axis_name="s")


@functools.partial(
    pl.kernel,
    out_type=jax.ShapeDtypeStruct((3 * 8 * _N,), jnp.int32),
    mesh=_sc_mesh,
    scratch_types=[
        pltpu.VMEM((_WIN,), jnp.int32),
        pltpu.VMEM((8 * _WIN,), jnp.int32),
    ],
    compiler_params=pltpu.CompilerParams(
        use_tc_tiling_on_sc=False, needs_layout_passes=False),
)
def _sc_inds(indsT_hbm, oindsT_hbm, ibuf, obuf):
    wid = lax.axis_index("s") * _NC + lax.axis_index("c")
    lane = lax.iota(jnp.int32, 16)
    sel = lane >> 3
    j = lane & 7

    for c in range(3):
        off = (_MASKS[c] >> j) & 1
        for t in range(4):
            g = wid + 32 * t

            @pl.when(g < _CHT)
            def _():
                pltpu.sync_copy(
                    indsT_hbm.at[pl.ds(c * _N + g * _WIN, _WIN)], ibuf)

                def step(p, carry):
                    v = plsc.load_gather(ibuf, [2 * p + sel])
                    obuf[pl.ds(16 * p, 16)] = 2 * v + off
                    return carry

                lax.fori_loop(0, _WIN // 2, step, 0)
                pltpu.sync_copy(
                    obuf,
                    oindsT_hbm.at[pl.ds(c * 8 * _N + g * 8 * _WIN, 8 * _WIN)])


_G0_NP = np.zeros((128, 1024), np.float32)
for _i in range(128):
    _G0_NP[_i, 8 * _i:8 * _i + 8] = 1.0


def _tc_body(g_ref, featsT_ref, ofeatsT_ref):
    x = featsT_ref[...]
    g = g_ref[...]
    for v in range(_K):
        xs = jax.lax.slice(x, (0, 128 * v), (64, 128 * v + 128))
        ofeatsT_ref[:, 1024 * v:1024 * (v + 1)] = jnp.dot(
            xs, g, preferred_element_type=jnp.float32)


def kernel(voxel_inds, feats):
    indsT_flat = voxel_inds.T.reshape(-1)
    featsT = feats.T

    oindsT_flat = _sc_inds(indsT_flat)
    ofeatsT = pl.pallas_call(
        _tc_body,
        grid=(_GRID,),
        in_specs=[pl.BlockSpec((128, 1024), lambda i: (0, 0)),
                  pl.BlockSpec((64, _LI), lambda i: (0, i))],
        out_specs=pl.BlockSpec((64, _LO), lambda i: (0, i)),
        out_shape=jax.ShapeDtypeStruct((64, 8 * _N), jnp.float32),
    )(jnp.asarray(_G0_NP), featsT)

    return oindsT_flat.reshape(3, 8 * _N).T, ofeatsT.T
